# baseline (speedup 1.0000x reference)
import functools

import jax
import jax.numpy as jnp
from jax import lax
from jax.experimental import pallas as pl
from jax.experimental.pallas import tpu as pltpu
from jax.experimental.pallas import tpu_sc as plsc

D = 128
B_ROWS = 4096
B_COLS = 50

_info = plsc.get_sparse_core_info()
NC = _info.num_cores
NS = _info.num_subcores
NW = NC * NS
SPAN = B_ROWS // NW
CHUNK = 64
NCHUNK = B_COLS * SPAN // CHUNK
NBUF = 10

_mesh = plsc.VectorSubcoreMesh(core_axis_name="c", subcore_axis_name="s")


@functools.partial(
    pl.kernel,
    mesh=_mesh,
    out_type=jax.ShapeDtypeStruct((B_COLS, B_ROWS, D), jnp.float32),
    scratch_types=[
        pltpu.VMEM((B_COLS, SPAN), jnp.int32),
        pltpu.VMEM((NBUF, CHUNK, D), jnp.float32),
        pltpu.SemaphoreType.DMA,
        pltpu.SemaphoreType.DMA,
    ],
)
def _embed_gather(ids_hbm, table_hbm, out_hbm, idx_v, rows_v, gsem, wsem):
    wid = lax.axis_index("s") * NC + lax.axis_index("c")
    i0 = wid * SPAN

    pltpu.sync_copy(ids_hbm.at[:, pl.ds(i0, SPAN)], idx_v)

    def idx_of(c):
        return idx_v.at[c // 2, pl.ds((c % 2) * CHUNK, CHUNK)]

    def dst_of(c):
        return out_hbm.at[c // 2, pl.ds(i0 + (c % 2) * CHUNK, CHUNK)]

    for b in range(NBUF):
        pltpu.async_copy(table_hbm.at[idx_of(b)], rows_v.at[b], gsem)

    def outer(g, carry):
        for b in range(NBUF):
            c = g * NBUF + b
            pltpu.make_async_copy(
                table_hbm.at[idx_of(c)], rows_v.at[b], gsem
            ).wait()
            wb = pltpu.make_async_copy(rows_v.at[b], dst_of(c), wsem)
            wb.start()

            @pl.when(c + NBUF < NCHUNK)
            def _():
                wb.wait()
                pltpu.async_copy(
                    table_hbm.at[idx_of(c + NBUF)], rows_v.at[b], gsem
                )
        return carry

    lax.fori_loop(0, NCHUNK // NBUF, outer, 0)

    for b in range(NBUF):
        c = NCHUNK - NBUF + b
        pltpu.make_async_copy(rows_v.at[b], dst_of(c), wsem).wait()


def kernel(text_ids, table):
    ids_t = text_ids.astype(jnp.int32).T
    out_t = _embed_gather(ids_t, table)
    return out_t.transpose(1, 0, 2)


# speedup vs baseline: 1.0046x; 1.0046x over previous
import functools

import jax
import jax.numpy as jnp
from jax import lax
from jax.experimental import pallas as pl
from jax.experimental.pallas import tpu as pltpu
from jax.experimental.pallas import tpu_sc as plsc

D = 128
B_ROWS = 4096
B_COLS = 50

_info = plsc.get_sparse_core_info()
NC = _info.num_cores
NS = _info.num_subcores
NW = NC * NS
SPAN = B_ROWS // NW
NBUF = 5

_mesh = plsc.VectorSubcoreMesh(core_axis_name="c", subcore_axis_name="s")


@functools.partial(
    pl.kernel,
    mesh=_mesh,
    out_type=jax.ShapeDtypeStruct((B_COLS, B_ROWS, D), jnp.float32),
    scratch_types=[
        pltpu.VMEM((B_COLS, SPAN), jnp.int32),
        pltpu.VMEM((NBUF, SPAN, D), jnp.float32),
        pltpu.SemaphoreType.DMA,
        pltpu.SemaphoreType.DMA,
        pltpu.SemaphoreType.DMA,
    ],
)
def _embed_gather(ids_hbm, table_hbm, out_hbm, idx_v, rows_v, gsem, wsem, ssem):
    wid = lax.axis_index("s") * NC + lax.axis_index("c")
    i0 = wid * SPAN

    LEAD = 8
    pltpu.sync_copy(
        ids_hbm.at[pl.ds(0, LEAD), pl.ds(i0, SPAN)],
        idx_v.at[pl.ds(0, LEAD)],
    )
    for b in range(NBUF):
        pltpu.async_copy(table_hbm.at[idx_v.at[b]], rows_v.at[b], gsem)
    rest = pltpu.make_async_copy(
        ids_hbm.at[pl.ds(LEAD, B_COLS - LEAD), pl.ds(i0, SPAN)],
        idx_v.at[pl.ds(LEAD, B_COLS - LEAD)],
        ssem,
    )
    rest.start()
    rest.wait()

    def outer(g, carry):
        for b in range(NBUF):
            j = g * NBUF + b
            pltpu.make_async_copy(
                table_hbm.at[idx_v.at[j]], rows_v.at[b], gsem
            ).wait()
            wb = pltpu.make_async_copy(
                rows_v.at[b],
                out_hbm.at[j, pl.ds(i0, SPAN)],
                wsem,
            )
            wb.start()

            @pl.when(j + NBUF < B_COLS)
            def _():
                wb.wait()
                pltpu.async_copy(
                    table_hbm.at[idx_v.at[j + NBUF]], rows_v.at[b], gsem
                )
        return carry

    lax.fori_loop(0, B_COLS // NBUF, outer, 0)

    for b in range(NBUF):
        j = B_COLS - NBUF + b
        pltpu.make_async_copy(
            rows_v.at[b],
            out_hbm.at[j, pl.ds(i0, SPAN)],
            wsem,
        ).wait()


def kernel(text_ids, table):
    ids_t = text_ids.astype(jnp.int32).T
    out_t = _embed_gather(ids_t, table)
    return out_t.transpose(1, 0, 2)
